# initial kernel scaffold (unmeasured)
import jax
import jax.numpy as jnp
from jax import lax
from jax.experimental import pallas as pl
from jax.experimental.pallas import tpu as pltpu

N_DEV = 4
B, SQ, HQ, DH = 2, 512, 8, 64
SKV_SHARD = 512
WIN = 128
D_MODEL = 768
D_QK = HQ * DH


def kernel(x, Wq, K_ext, V_ext, Wo):
    def body(x_ref, wq_ref, k_ref, v_ref, wo_ref, out_ref,
             ctx_ref, k1_ref, v1_ref,
             snd_kv, rcv_kv, snd_ctx, rcv_ctx):
        my = lax.axis_index("i")

        barrier = pltpu.get_barrier_semaphore()
        for other in range(N_DEV):
            if True:
                pass
        for other in (1, 2, 3):
            pass
        my_ = my
        for nbr in range(N_DEV):
            pl.semaphore_signal(
                barrier, inc=1,
                device_id=(nbr,), device_id_type=pl.DeviceIdType.MESH,
            )
        pl.semaphore_wait(barrier, N_DEV)

        @pl.when(my == 1)
        def _():
            k1_ref[...] = k_ref[:, 0:WIN, :, :]
            v1_ref[...] = v_ref[:, 0:WIN, :, :]
            rk = pltpu.make_async_remote_copy(
                src_ref=k1_ref, dst_ref=k1_ref,
                send_sem=snd_kv.at[0], recv_sem=rcv_kv.at[0],
                device_id=(0,), device_id_type=pl.DeviceIdType.MESH,
            )
            rv = pltpu.make_async_remote_copy(
                src_ref=v1_ref, dst_ref=v1_ref,
                send_sem=snd_kv.at[1], recv_sem=rcv_kv.at[1],
                device_id=(0,), device_id_type=pl.DeviceIdType.MESH,
            )
            rk.start()
            rv.start()
            rk.wait_send()
            rv.wait_send()

        @pl.when(my == 0)
        def _():
            rk = pltpu.make_async_remote_copy(
                src_ref=k1_ref, dst_ref=k1_ref,
                send_sem=snd_kv.at[0], recv_sem=rcv_kv.at[0],
                device_id=(1,), device_id_type=pl.DeviceIdType.MESH,
            )
            rv = pltpu.make_async_remote_copy(
                src_ref=v1_ref, dst_ref=v1_ref,
                send_sem=snd_kv.at[1], recv_sem=rcv_kv.at[1],
                device_id=(1,), device_id_type=pl.DeviceIdType.MESH,
            )
            rk.wait_recv()
            rv.wait_recv()

            qI = lax.broadcasted_iota(jnp.int32, (SQ, SKV_SHARD), 0)
            kI = lax.broadcasted_iota(jnp.int32, (SQ, SKV_SHARD), 1)
            mask_a = jnp.abs(qI - kI) <= WIN
            qIb = lax.broadcasted_iota(jnp.int32, (SQ, WIN), 0)
            kIb = lax.broadcasted_iota(jnp.int32, (SQ, WIN), 1) + SKV_SHARD
            mask_b = (kIb - qIb) <= WIN

            for b in range(B):
                qb = jnp.dot(x_ref[b], wq_ref[...],
                             preferred_element_type=jnp.float32)
                kb = k_ref[b].reshape(SKV_SHARD, D_QK)
                vb = v_ref[b].reshape(SKV_SHARD, D_QK)
                k1b = k1_ref[b].reshape(WIN, D_QK)
                v1b = v1_ref[b].reshape(WIN, D_QK)
                for h in range(HQ):
                    sl = slice(h * DH, (h + 1) * DH)
                    qh = qb[:, sl]
                    sa = lax.dot_general(
                        qh, kb[:, sl], (((1,), (1,)), ((), ())),
                        preferred_element_type=jnp.float32)
                    sb = lax.dot_general(
                        qh, k1b[:, sl], (((1,), (1,)), ((), ())),
                        preferred_element_type=jnp.float32)
                    sa = jnp.where(mask_a, sa * 0.125, -1e9)
                    sb = jnp.where(mask_b, sb * 0.125, -1e9)
                    m = jnp.maximum(jnp.max(sa, axis=1, keepdims=True),
                                    jnp.max(sb, axis=1, keepdims=True))
                    ea = jnp.exp(sa - m)
                    eb = jnp.exp(sb - m)
                    l = (jnp.sum(ea, axis=1, keepdims=True)
                         + jnp.sum(eb, axis=1, keepdims=True))
                    acc = (jnp.dot(ea, vb[:, sl],
                                   preferred_element_type=jnp.float32)
                           + jnp.dot(eb, v1b[:, sl],
                                     preferred_element_type=jnp.float32))
                    ctx_ref[b, :, sl] = acc / l

            for t in (1, 2, 3):
                send = pltpu.make_async_remote_copy(
                    src_ref=ctx_ref, dst_ref=ctx_ref,
                    send_sem=snd_ctx.at[t - 1], recv_sem=rcv_ctx.at[0],
                    device_id=(t,), device_id_type=pl.DeviceIdType.MESH,
                )
                send.start()
            for t in (1, 2, 3):
                done = pltpu.make_async_remote_copy(
                    src_ref=ctx_ref, dst_ref=ctx_ref,
                    send_sem=snd_ctx.at[t - 1], recv_sem=rcv_ctx.at[0],
                    device_id=(t,), device_id_type=pl.DeviceIdType.MESH,
                )
                done.wait_send()

        @pl.when(my != 0)
        def _():
            recv = pltpu.make_async_remote_copy(
                src_ref=ctx_ref, dst_ref=ctx_ref,
                send_sem=snd_ctx.at[0], recv_sem=rcv_ctx.at[0],
                device_id=(0,), device_id_type=pl.DeviceIdType.MESH,
            )
            recv.wait_recv()

        for b in range(B):
            out_ref[b] = jnp.dot(ctx_ref[b], wo_ref[...],
                                 preferred_element_type=jnp.float32)

    return pl.pallas_call(
        body,
        out_shape=jax.ShapeDtypeStruct((B, SQ, D_MODEL), jnp.float32),
        in_specs=[pl.BlockSpec(memory_space=pltpu.VMEM)] * 5,
        out_specs=pl.BlockSpec(memory_space=pltpu.VMEM),
        scratch_shapes=[
            pltpu.VMEM((B, SQ, D_QK), jnp.float32),
            pltpu.VMEM((B, WIN, HQ, DH), jnp.float32),
            pltpu.VMEM((B, WIN, HQ, DH), jnp.float32),
            pltpu.SemaphoreType.DMA((2,)),
            pltpu.SemaphoreType.DMA((2,)),
            pltpu.SemaphoreType.DMA((3,)),
            pltpu.SemaphoreType.DMA((1,)),
        ],
        compiler_params=pltpu.CompilerParams(
            collective_id=0,
            vmem_limit_bytes=100 * 1024 * 1024,
        ),
    )(x, Wq, K_ext, V_ext, Wo)


# baseline (device time: 106869 ns/iter reference)
import jax
import jax.numpy as jnp
from jax import lax
from jax.experimental import pallas as pl
from jax.experimental.pallas import tpu as pltpu

N_DEV = 4
B, SQ, HQ, DH = 2, 512, 8, 64
SKV_SHARD = 512
WIN = 128
D_MODEL = 768
D_QK = HQ * DH


def kernel(x, Wq, K_ext, V_ext, Wo):
    def body(x_ref, wq_ref, k_ref, v_ref, wo_ref, out_ref,
             ctx_ref, k1_ref, v1_ref,
             snd_kv, rcv_kv, snd_ctx, rcv_ctx):
        my = lax.axis_index("i")

        barrier = pltpu.get_barrier_semaphore()
        for nbr in range(N_DEV):
            pl.semaphore_signal(
                barrier, inc=1,
                device_id=(nbr,), device_id_type=pl.DeviceIdType.MESH,
            )
        pl.semaphore_wait(barrier, N_DEV)

        @pl.when(my == 1)
        def _():
            k1_ref[...] = k_ref[:, 0:WIN, :, :]
            v1_ref[...] = v_ref[:, 0:WIN, :, :]
            rk = pltpu.make_async_remote_copy(
                src_ref=k1_ref, dst_ref=k1_ref,
                send_sem=snd_kv.at[0], recv_sem=rcv_kv.at[0],
                device_id=(0,), device_id_type=pl.DeviceIdType.MESH,
            )
            rv = pltpu.make_async_remote_copy(
                src_ref=v1_ref, dst_ref=v1_ref,
                send_sem=snd_kv.at[1], recv_sem=rcv_kv.at[1],
                device_id=(0,), device_id_type=pl.DeviceIdType.MESH,
            )
            rk.start()
            rv.start()
            rk.wait_send()
            rv.wait_send()

        @pl.when(my == 0)
        def _():
            rk = pltpu.make_async_remote_copy(
                src_ref=k1_ref, dst_ref=k1_ref,
                send_sem=snd_kv.at[0], recv_sem=rcv_kv.at[0],
                device_id=(1,), device_id_type=pl.DeviceIdType.MESH,
            )
            rv = pltpu.make_async_remote_copy(
                src_ref=v1_ref, dst_ref=v1_ref,
                send_sem=snd_kv.at[1], recv_sem=rcv_kv.at[1],
                device_id=(1,), device_id_type=pl.DeviceIdType.MESH,
            )
            rk.wait_recv()
            rv.wait_recv()

            qI = lax.broadcasted_iota(jnp.int32, (SQ, SKV_SHARD), 0)
            kI = lax.broadcasted_iota(jnp.int32, (SQ, SKV_SHARD), 1)
            mask_a = jnp.abs(qI - kI) <= WIN
            qIb = lax.broadcasted_iota(jnp.int32, (SQ, WIN), 0)
            kIb = lax.broadcasted_iota(jnp.int32, (SQ, WIN), 1) + SKV_SHARD
            mask_b = (kIb - qIb) <= WIN

            for b in range(B):
                qb = jnp.dot(x_ref[b], wq_ref[...],
                             preferred_element_type=jnp.float32)
                kb = k_ref[b].reshape(SKV_SHARD, D_QK)
                vb = v_ref[b].reshape(SKV_SHARD, D_QK)
                k1b = k1_ref[b].reshape(WIN, D_QK)
                v1b = v1_ref[b].reshape(WIN, D_QK)
                for h in range(HQ):
                    sl = slice(h * DH, (h + 1) * DH)
                    qh = qb[:, sl]
                    sa = lax.dot_general(
                        qh, kb[:, sl], (((1,), (1,)), ((), ())),
                        preferred_element_type=jnp.float32)
                    sb = lax.dot_general(
                        qh, k1b[:, sl], (((1,), (1,)), ((), ())),
                        preferred_element_type=jnp.float32)
                    sa = jnp.where(mask_a, sa * 0.125, -1e9)
                    sb = jnp.where(mask_b, sb * 0.125, -1e9)
                    m = jnp.maximum(jnp.max(sa, axis=1, keepdims=True),
                                    jnp.max(sb, axis=1, keepdims=True))
                    ea = jnp.exp(sa - m)
                    eb = jnp.exp(sb - m)
                    l = (jnp.sum(ea, axis=1, keepdims=True)
                         + jnp.sum(eb, axis=1, keepdims=True))
                    acc = (jnp.dot(ea, vb[:, sl],
                                   preferred_element_type=jnp.float32)
                           + jnp.dot(eb, v1b[:, sl],
                                     preferred_element_type=jnp.float32))
                    ctx_ref[b, :, sl] = acc / l

            for t in (1, 2, 3):
                send = pltpu.make_async_remote_copy(
                    src_ref=ctx_ref, dst_ref=ctx_ref,
                    send_sem=snd_ctx.at[t - 1], recv_sem=rcv_ctx.at[0],
                    device_id=(t,), device_id_type=pl.DeviceIdType.MESH,
                )
                send.start()
            for t in (1, 2, 3):
                done = pltpu.make_async_remote_copy(
                    src_ref=ctx_ref, dst_ref=ctx_ref,
                    send_sem=snd_ctx.at[t - 1], recv_sem=rcv_ctx.at[0],
                    device_id=(t,), device_id_type=pl.DeviceIdType.MESH,
                )
                done.wait_send()

        @pl.when(my != 0)
        def _():
            recv = pltpu.make_async_remote_copy(
                src_ref=ctx_ref, dst_ref=ctx_ref,
                send_sem=snd_ctx.at[0], recv_sem=rcv_ctx.at[0],
                device_id=(0,), device_id_type=pl.DeviceIdType.MESH,
            )
            recv.wait_recv()

        for b in range(B):
            out_ref[b] = jnp.dot(ctx_ref[b], wo_ref[...],
                                 preferred_element_type=jnp.float32)

    return pl.pallas_call(
        body,
        out_shape=jax.ShapeDtypeStruct((B, SQ, D_MODEL), jnp.float32),
        in_specs=[pl.BlockSpec(memory_space=pltpu.VMEM)] * 5,
        out_specs=pl.BlockSpec(memory_space=pltpu.VMEM),
        scratch_shapes=[
            pltpu.VMEM((B, SQ, D_QK), jnp.float32),
            pltpu.VMEM((B, WIN, HQ, DH), jnp.float32),
            pltpu.VMEM((B, WIN, HQ, DH), jnp.float32),
            pltpu.SemaphoreType.DMA((2,)),
            pltpu.SemaphoreType.DMA((2,)),
            pltpu.SemaphoreType.DMA((3,)),
            pltpu.SemaphoreType.DMA((1,)),
        ],
        compiler_params=pltpu.CompilerParams(
            collective_id=0,
            vmem_limit_bytes=100 * 1024 * 1024,
        ),
    )(x, Wq, K_ext, V_ext, Wo)


# device time: 78389 ns/iter; 1.3633x vs baseline; 1.3633x over previous
import jax
import jax.numpy as jnp
from jax import lax
from jax.experimental import pallas as pl
from jax.experimental.pallas import tpu as pltpu

N_DEV = 4
B, SQ, HQ, DH = 2, 512, 8, 64
SKV_SHARD = 512
WIN = 128
D_MODEL = 768
D_QK = HQ * DH
HGRP = HQ // 2
CW = HGRP * DH
N_CHUNK = B * 2


def kernel(x, Wq, K_ext, V_ext, Wo):
    def body(x_ref, wq_ref, k_ref, v_ref, wo_ref, out_ref,
             ctx_ref, k1_ref, v1_ref,
             snd_kv, rcv_kv, snd_ctx, snd_fwd, rcv_ctx):
        my = lax.axis_index("i")

        barrier = pltpu.get_barrier_semaphore()
        for nbr in range(N_DEV):
            pl.semaphore_signal(
                barrier, inc=1,
                device_id=(nbr,), device_id_type=pl.DeviceIdType.MESH,
            )
        pl.semaphore_wait(barrier, N_DEV)

        def kv_rdma(slot, buf, peer):
            return pltpu.make_async_remote_copy(
                src_ref=buf, dst_ref=buf,
                send_sem=snd_kv.at[slot], recv_sem=rcv_kv.at[slot],
                device_id=(peer,), device_id_type=pl.DeviceIdType.MESH,
            )

        def chunk_rdma(c, sem_arr, slot, peer):
            return pltpu.make_async_remote_copy(
                src_ref=ctx_ref.at[c], dst_ref=ctx_ref.at[c],
                send_sem=sem_arr.at[slot], recv_sem=rcv_ctx.at[c],
                device_id=(peer,), device_id_type=pl.DeviceIdType.MESH,
            )

        def accum_out(c):
            b, g = divmod(c, 2)
            cols = slice(g * CW, (g + 1) * CW)
            part = jnp.dot(ctx_ref[c], wo_ref[cols, :],
                           preferred_element_type=jnp.float32)
            if g == 0:
                out_ref[b] = part
            else:
                out_ref[b] = out_ref[b] + part

        @pl.when(my == 1)
        def _():
            k1_ref[...] = k_ref[:, 0:WIN, :, :]
            v1_ref[...] = v_ref[:, 0:WIN, :, :]
            rk = kv_rdma(0, k1_ref, 0)
            rv = kv_rdma(1, v1_ref, 0)
            rk.start()
            rv.start()
            for c in range(N_CHUNK):
                chunk_rdma(c, snd_ctx, 2 * c, 0).wait_recv()
                chunk_rdma(c, snd_fwd, c, 2).start()
                accum_out(c)
            rk.wait_send()
            rv.wait_send()
            for c in range(N_CHUNK):
                chunk_rdma(c, snd_fwd, c, 2).wait_send()

        @pl.when(my == 0)
        def _():
            qs = [jnp.dot(x_ref[b], wq_ref[...],
                          preferred_element_type=jnp.float32)
                  for b in range(B)]

            rk = kv_rdma(0, k1_ref, 1)
            rv = kv_rdma(1, v1_ref, 1)
            rk.wait_recv()
            rv.wait_recv()

            qI = lax.broadcasted_iota(jnp.int32, (SQ, SKV_SHARD), 0)
            kI = lax.broadcasted_iota(jnp.int32, (SQ, SKV_SHARD), 1)
            mask_a = jnp.abs(qI - kI) <= WIN
            qIb = lax.broadcasted_iota(jnp.int32, (SQ, WIN), 0)
            kIb = lax.broadcasted_iota(jnp.int32, (SQ, WIN), 1) + SKV_SHARD
            mask_b = (kIb - qIb) <= WIN

            for c in range(N_CHUNK):
                b, g = divmod(c, 2)
                kb = k_ref[b].reshape(SKV_SHARD, D_QK)
                vb = v_ref[b].reshape(SKV_SHARD, D_QK)
                k1b = k1_ref[b].reshape(WIN, D_QK)
                v1b = v1_ref[b].reshape(WIN, D_QK)
                for hh in range(HGRP):
                    h = g * HGRP + hh
                    sl = slice(h * DH, (h + 1) * DH)
                    qh = qs[b][:, sl]
                    sa = lax.dot_general(
                        qh, kb[:, sl], (((1,), (1,)), ((), ())),
                        preferred_element_type=jnp.float32)
                    sb = lax.dot_general(
                        qh, k1b[:, sl], (((1,), (1,)), ((), ())),
                        preferred_element_type=jnp.float32)
                    sa = jnp.where(mask_a, sa * 0.125, -1e9)
                    sb = jnp.where(mask_b, sb * 0.125, -1e9)
                    m = jnp.maximum(jnp.max(sa, axis=1, keepdims=True),
                                    jnp.max(sb, axis=1, keepdims=True))
                    ea = jnp.exp(sa - m)
                    eb = jnp.exp(sb - m)
                    l = (jnp.sum(ea, axis=1, keepdims=True)
                         + jnp.sum(eb, axis=1, keepdims=True))
                    acc = (jnp.dot(ea, vb[:, sl],
                                   preferred_element_type=jnp.float32)
                           + jnp.dot(eb, v1b[:, sl],
                                     preferred_element_type=jnp.float32))
                    ctx_ref[c, :, hh * DH:(hh + 1) * DH] = acc / l
                chunk_rdma(c, snd_ctx, 2 * c, 1).start()
                chunk_rdma(c, snd_ctx, 2 * c + 1, 3).start()
                accum_out(c)

            for c in range(N_CHUNK):
                chunk_rdma(c, snd_ctx, 2 * c, 1).wait_send()
                chunk_rdma(c, snd_ctx, 2 * c + 1, 3).wait_send()

        @pl.when(my == 3)
        def _():
            for c in range(N_CHUNK):
                chunk_rdma(c, snd_ctx, 2 * c + 1, 0).wait_recv()
                accum_out(c)

        @pl.when(my == 2)
        def _():
            for c in range(N_CHUNK):
                chunk_rdma(c, snd_fwd, c, 1).wait_recv()
                accum_out(c)

    return pl.pallas_call(
        body,
        out_shape=jax.ShapeDtypeStruct((B, SQ, D_MODEL), jnp.float32),
        in_specs=[pl.BlockSpec(memory_space=pltpu.VMEM)] * 5,
        out_specs=pl.BlockSpec(memory_space=pltpu.VMEM),
        scratch_shapes=[
            pltpu.VMEM((N_CHUNK, SQ, CW), jnp.float32),
            pltpu.VMEM((B, WIN, HQ, DH), jnp.float32),
            pltpu.VMEM((B, WIN, HQ, DH), jnp.float32),
            pltpu.SemaphoreType.DMA((2,)),
            pltpu.SemaphoreType.DMA((2,)),
            pltpu.SemaphoreType.DMA((2 * N_CHUNK,)),
            pltpu.SemaphoreType.DMA((N_CHUNK,)),
            pltpu.SemaphoreType.DMA((N_CHUNK,)),
        ],
        compiler_params=pltpu.CompilerParams(
            collective_id=0,
            vmem_limit_bytes=100 * 1024 * 1024,
        ),
    )(x, Wq, K_ext, V_ext, Wo)
